# baseline (speedup 1.0000x reference)
import functools

import jax
import jax.numpy as jnp
from jax import lax
from jax.experimental import pallas as pl
from jax.experimental.pallas import tpu as pltpu
from jax.experimental.pallas import tpu_sc as plsc

CUTOFF2 = 25.0
K = 32


def _topk_body(centers_ref, posT_ref, cvalid_ref, vvalidT_ref,
               src_ref, w_ref, ok_ref, *, R, P):
    n = pl.program_id(0)
    b = pl.program_id(1)

    cx = centers_ref[0, :, 0:1]
    cy = centers_ref[0, :, 1:2]
    cz = centers_ref[0, :, 2:3]
    ax = posT_ref[0, 0:1, :]
    ay = posT_ref[0, 1:2, :]
    az = posT_ref[0, 2:3, :]

    dx = cx - ax
    dy = cy - ay
    dz = cz - az
    d = (dx * dx + dy * dy) + dz * dz

    col = lax.broadcasted_iota(jnp.int32, (R, P), 1)
    rowg = b * R + lax.broadcasted_iota(jnp.int32, (R, P), 0)
    okpair = ((d <= CUTOFF2) & (col != rowg)
              & (cvalid_ref[0, :, 0:1] > 0.0) & (vvalidT_ref[0, 0:1, :] > 0.0))
    d = jnp.where(okpair, d, jnp.inf)

    base = n * P
    centerg = base + b * R + lax.broadcasted_iota(jnp.int32, (R, 1), 0)

    for k in range(K):
        m = jnp.min(d, axis=1, keepdims=True)
        cand = jnp.where(d == m, col, P)
        j = jnp.min(cand, axis=1, keepdims=True)
        okk = m < jnp.inf
        d = jnp.where(col == j, jnp.inf, d)
        src_ref[0, :, k:k + 1] = jnp.where(okk, j + base, centerg)
        w_ref[0, :, k:k + 1] = jnp.where(okk, jnp.sqrt(m), 0.0)
        ok_ref[0, :, k:k + 1] = okk.astype(jnp.float32)


@functools.partial(jax.jit, static_argnames=("interpret",))
def _run_topk(pos, posT, cvalid, vvalidT, interpret=False):
    N, P, _ = pos.shape
    R = 896
    NB = P // R
    o = jax.ShapeDtypeStruct((N, P, K), jnp.float32)
    oi = jax.ShapeDtypeStruct((N, P, K), jnp.int32)
    return pl.pallas_call(
        functools.partial(_topk_body, R=R, P=P),
        grid=(N, NB),
        in_specs=[
            pl.BlockSpec((1, R, 3), lambda n, b: (n, b, 0)),
            pl.BlockSpec((1, 3, P), lambda n, b: (n, 0, 0)),
            pl.BlockSpec((1, R, 1), lambda n, b: (n, b, 0)),
            pl.BlockSpec((1, 1, P), lambda n, b: (n, 0, 0)),
        ],
        out_specs=[pl.BlockSpec((1, R, K), lambda n, b: (n, b, 0))] * 3,
        out_shape=[oi, o, o],
        interpret=interpret,
    )(pos, posT, cvalid, vvalidT)


def _edge_vec_sc(px, py, pz, src, dst):
    NP = px.shape[0]
    E = src.shape[0]
    info = plsc.get_sparse_core_info()
    NC, NS, L = info.num_cores, info.num_subcores, info.num_lanes
    NW = NC * NS
    epw = E // NW
    mesh = plsc.VectorSubcoreMesh(core_axis_name="c", subcore_axis_name="s")
    of = jax.ShapeDtypeStruct((E,), jnp.float32)

    @functools.partial(
        pl.kernel, mesh=mesh,
        out_type=(of, of, of),
        compiler_params=pltpu.CompilerParams(needs_layout_passes=False),
        scratch_types=[
            pltpu.VMEM((NP,), jnp.float32),
            pltpu.VMEM((NP,), jnp.float32),
            pltpu.VMEM((NP,), jnp.float32),
            pltpu.VMEM((epw,), jnp.int32),
            pltpu.VMEM((epw,), jnp.int32),
            pltpu.VMEM((epw,), jnp.float32),
            pltpu.VMEM((epw,), jnp.float32),
            pltpu.VMEM((epw,), jnp.float32),
        ],
    )
    def gather_kernel(px_hbm, py_hbm, pz_hbm, src_hbm, dst_hbm,
                      ox_hbm, oy_hbm, oz_hbm,
                      px_v, py_v, pz_v, src_v, dst_v, ox_v, oy_v, oz_v):
        wid = lax.axis_index("s") * NC + lax.axis_index("c")
        base = wid * epw
        pltpu.sync_copy(px_hbm, px_v)
        pltpu.sync_copy(py_hbm, py_v)
        pltpu.sync_copy(pz_hbm, pz_v)
        pltpu.sync_copy(src_hbm.at[pl.ds(base, epw)], src_v)
        pltpu.sync_copy(dst_hbm.at[pl.ds(base, epw)], dst_v)

        def body(i, carry):
            sl = pl.ds(i * L, L)
            s = src_v[sl]
            t = dst_v[sl]
            ox_v[sl] = plsc.load_gather(px_v, [s]) - plsc.load_gather(px_v, [t])
            oy_v[sl] = plsc.load_gather(py_v, [s]) - plsc.load_gather(py_v, [t])
            oz_v[sl] = plsc.load_gather(pz_v, [s]) - plsc.load_gather(pz_v, [t])
            return carry

        lax.fori_loop(0, epw // L, body, 0)
        pltpu.sync_copy(ox_v, ox_hbm.at[pl.ds(base, epw)])
        pltpu.sync_copy(oy_v, oy_hbm.at[pl.ds(base, epw)])
        pltpu.sync_copy(oz_v, oz_hbm.at[pl.ds(base, epw)])

    return gather_kernel(px, py, pz, src, dst)


def kernel(pos_atoms, mask_atoms):
    N, L_, A, _ = pos_atoms.shape
    P = L_ * A
    pos = jnp.where(mask_atoms[..., None], pos_atoms, 0.0).reshape(N, P, 3)
    validf = mask_atoms.reshape(N, P).astype(jnp.float32)
    posT = jnp.transpose(pos, (0, 2, 1))
    src3, w3, ok3 = _run_topk(pos, posT, validf[..., None], validf[:, None, :])
    src = src3.reshape(-1)
    dst = jnp.broadcast_to(jnp.arange(N * P, dtype=jnp.int32)[:, None],
                           (N * P, K)).reshape(-1)
    edge_index = jnp.stack([src, dst], axis=0)
    edge_weight = w3.reshape(-1)
    ok = ok3.reshape(-1) > 0.0
    px = posT[:, 0, :].reshape(-1)
    py = posT[:, 1, :].reshape(-1)
    pz = posT[:, 2, :].reshape(-1)
    ex, ey, ez = _edge_vec_sc(px, py, pz, src, dst)
    edge_vec = jnp.stack([ex, ey, ez], axis=-1)
    return edge_index, edge_weight, edge_vec, ok


# speedup vs baseline: 1.4190x; 1.4190x over previous
import functools

import jax
import jax.numpy as jnp
from jax import lax
from jax.experimental import pallas as pl
from jax.experimental.pallas import tpu as pltpu
from jax.experimental.pallas import tpu_sc as plsc

CUTOFF2 = 25.0
K = 32


def _topk_body(centers_ref, posT_ref, cvalid_ref, vvalidT_ref,
               src_ref, w_ref, ok_ref, *, R, P):
    n = pl.program_id(0)
    b = pl.program_id(1)

    cx = centers_ref[0, :, 0:1]
    cy = centers_ref[0, :, 1:2]
    cz = centers_ref[0, :, 2:3]
    ax = posT_ref[0, 0:1, :]
    ay = posT_ref[0, 1:2, :]
    az = posT_ref[0, 2:3, :]

    dx = cx - ax
    dy = cy - ay
    dz = cz - az
    d = (dx * dx + dy * dy) + dz * dz

    colf = lax.broadcasted_iota(jnp.int32, (R, P), 1).astype(jnp.float32)
    rowgf = (b * R + lax.broadcasted_iota(jnp.int32, (R, P), 0)).astype(jnp.float32)
    okpair = ((d <= CUTOFF2) & (colf != rowgf)
              & (cvalid_ref[0, :, 0:1] > 0.0) & (vvalidT_ref[0, 0:1, :] > 0.0))
    d = jnp.where(okpair, d, jnp.inf)

    base = n * P
    centerg = base + b * R + lax.broadcasted_iota(jnp.int32, (R, 1), 0)
    bigf = jnp.float32(P)

    for k in range(K):
        m = jnp.min(d, axis=1, keepdims=True)
        cand = jnp.where(d == m, colf, bigf)
        jf = jnp.min(cand, axis=1, keepdims=True)
        okk = m < jnp.inf
        d = jnp.where(colf == jf, jnp.inf, d)
        j = jf.astype(jnp.int32)
        src_ref[0, :, k:k + 1] = jnp.where(okk, j + base, centerg)
        w_ref[0, :, k:k + 1] = jnp.where(okk, jnp.sqrt(m), 0.0)
        ok_ref[0, :, k:k + 1] = okk.astype(jnp.float32)


@functools.partial(jax.jit, static_argnames=("interpret",))
def _run_topk(pos, posT, cvalid, vvalidT, interpret=False):
    N, P, _ = pos.shape
    R = 448
    NB = P // R
    o = jax.ShapeDtypeStruct((N, P, K), jnp.float32)
    oi = jax.ShapeDtypeStruct((N, P, K), jnp.int32)
    return pl.pallas_call(
        functools.partial(_topk_body, R=R, P=P),
        grid=(N, NB),
        in_specs=[
            pl.BlockSpec((1, R, 3), lambda n, b: (n, b, 0)),
            pl.BlockSpec((1, 3, P), lambda n, b: (n, 0, 0)),
            pl.BlockSpec((1, R, 1), lambda n, b: (n, b, 0)),
            pl.BlockSpec((1, 1, P), lambda n, b: (n, 0, 0)),
        ],
        out_specs=[pl.BlockSpec((1, R, K), lambda n, b: (n, b, 0))] * 3,
        out_shape=[oi, o, o],
        interpret=interpret,
    )(pos, posT, cvalid, vvalidT)


def _edge_vec_sc(px, py, pz, src, dst):
    NP = px.shape[0]
    E = src.shape[0]
    info = plsc.get_sparse_core_info()
    NC, NS, L = info.num_cores, info.num_subcores, info.num_lanes
    NW = NC * NS
    epw = E // NW
    mesh = plsc.VectorSubcoreMesh(core_axis_name="c", subcore_axis_name="s")
    of = jax.ShapeDtypeStruct((E,), jnp.float32)

    @functools.partial(
        pl.kernel, mesh=mesh,
        out_type=(of, of, of),
        compiler_params=pltpu.CompilerParams(needs_layout_passes=False),
        scratch_types=[
            pltpu.VMEM((NP,), jnp.float32),
            pltpu.VMEM((NP,), jnp.float32),
            pltpu.VMEM((NP,), jnp.float32),
            pltpu.VMEM((epw,), jnp.int32),
            pltpu.VMEM((epw,), jnp.int32),
            pltpu.VMEM((epw,), jnp.float32),
            pltpu.VMEM((epw,), jnp.float32),
            pltpu.VMEM((epw,), jnp.float32),
        ],
    )
    def gather_kernel(px_hbm, py_hbm, pz_hbm, src_hbm, dst_hbm,
                      ox_hbm, oy_hbm, oz_hbm,
                      px_v, py_v, pz_v, src_v, dst_v, ox_v, oy_v, oz_v):
        wid = lax.axis_index("s") * NC + lax.axis_index("c")
        base = wid * epw
        pltpu.sync_copy(px_hbm, px_v)
        pltpu.sync_copy(py_hbm, py_v)
        pltpu.sync_copy(pz_hbm, pz_v)
        pltpu.sync_copy(src_hbm.at[pl.ds(base, epw)], src_v)
        pltpu.sync_copy(dst_hbm.at[pl.ds(base, epw)], dst_v)

        def body(i, carry):
            sl = pl.ds(i * L, L)
            s = src_v[sl]
            t = dst_v[sl]
            ox_v[sl] = plsc.load_gather(px_v, [s]) - plsc.load_gather(px_v, [t])
            oy_v[sl] = plsc.load_gather(py_v, [s]) - plsc.load_gather(py_v, [t])
            oz_v[sl] = plsc.load_gather(pz_v, [s]) - plsc.load_gather(pz_v, [t])
            return carry

        lax.fori_loop(0, epw // L, body, 0)
        pltpu.sync_copy(ox_v, ox_hbm.at[pl.ds(base, epw)])
        pltpu.sync_copy(oy_v, oy_hbm.at[pl.ds(base, epw)])
        pltpu.sync_copy(oz_v, oz_hbm.at[pl.ds(base, epw)])

    return gather_kernel(px, py, pz, src, dst)


def kernel(pos_atoms, mask_atoms):
    N, L_, A, _ = pos_atoms.shape
    P = L_ * A
    pos = jnp.where(mask_atoms[..., None], pos_atoms, 0.0).reshape(N, P, 3)
    validf = mask_atoms.reshape(N, P).astype(jnp.float32)
    posT = jnp.transpose(pos, (0, 2, 1))
    src3, w3, ok3 = _run_topk(pos, posT, validf[..., None], validf[:, None, :])
    src = src3.reshape(-1)
    dst = jnp.broadcast_to(jnp.arange(N * P, dtype=jnp.int32)[:, None],
                           (N * P, K)).reshape(-1)
    edge_index = jnp.stack([src, dst], axis=0)
    edge_weight = w3.reshape(-1)
    ok = ok3.reshape(-1) > 0.0
    px = posT[:, 0, :].reshape(-1)
    py = posT[:, 1, :].reshape(-1)
    pz = posT[:, 2, :].reshape(-1)
    ex, ey, ez = _edge_vec_sc(px, py, pz, src, dst)
    edge_vec = jnp.stack([ex, ey, ez], axis=-1)
    return edge_index, edge_weight, edge_vec, ok
